# speedup vs baseline: 16.6943x; 16.6943x over previous
import functools

import jax
import jax.numpy as jnp
from jax import lax
from jax.experimental import pallas as pl
from jax.experimental.pallas import tpu as pltpu
from jax.experimental.pallas import tpu_sc as plsc

N_VERTEX = 50000
NU = 8
D_LAT = 128
N_ANCIL = 8
ROWS = N_VERTEX * NU
CHUNK = 800
NCHUNK = ROWS // CHUNK


def _sc_gather(table, idx_flat):
    info = plsc.get_sparse_core_info()
    nc, ns = info.num_cores, info.num_subcores
    nw = nc * ns
    kmax = -(-NCHUNK // nw)

    mesh = plsc.VectorSubcoreMesh(core_axis_name="c", subcore_axis_name="s")

    @functools.partial(
        pl.kernel,
        out_type=jax.ShapeDtypeStruct((ROWS, D_LAT), jnp.float32),
        mesh=mesh,
        scratch_types=[
            pltpu.VMEM((CHUNK,), jnp.int32),
            pltpu.VMEM((CHUNK, D_LAT), jnp.float32),
            pltpu.SemaphoreType.DMA,
        ],
    )
    def gather_kernel(table_hbm, idx_hbm, out_hbm, idx_v, rows_v, sem):
        wid = lax.axis_index("s") * nc + lax.axis_index("c")

        def body(j, carry):
            c = j * nw + wid

            @pl.when(c < NCHUNK)
            def _():
                base = c * CHUNK
                pltpu.sync_copy(idx_hbm.at[pl.ds(base, CHUNK)], idx_v)
                pltpu.async_copy(table_hbm.at[idx_v], rows_v, sem).wait()
                pltpu.sync_copy(rows_v, out_hbm.at[pl.ds(base, CHUNK)])

            return carry

        lax.fori_loop(0, kmax, body, 0)

    return gather_kernel(table, idx_flat)


def _tc_transpose(x):
    def tkernel(x_ref, o_ref):
        o_ref[...] = x_ref[...].T

    return pl.pallas_call(
        tkernel,
        out_shape=jax.ShapeDtypeStruct((N_VERTEX, N_ANCIL), jnp.float32),
    )(x)


def kernel(z_prime, x_ancil, index):
    idx_flat = index.reshape(ROWS).astype(jnp.int32)
    z_rows = _sc_gather(z_prime, idx_flat)
    z_tilde = z_rows.reshape(N_VERTEX, NU, D_LAT)
    x_ancil_tilde = _tc_transpose(x_ancil)
    return z_tilde, x_ancil_tilde


# speedup vs baseline: 17.7211x; 1.0615x over previous
import functools

import jax
import jax.numpy as jnp
from jax import lax
from jax.experimental import pallas as pl
from jax.experimental.pallas import tpu as pltpu
from jax.experimental.pallas import tpu_sc as plsc

N_VERTEX = 50000
NU = 8
D_LAT = 128
N_ANCIL = 8
ROWS = N_VERTEX * NU
CHUNK = 400
NCHUNK = ROWS // CHUNK
NBUF = 2


def _sc_gather(table, idx_flat):
    info = plsc.get_sparse_core_info()
    nc, ns = info.num_cores, info.num_subcores
    nw = nc * ns
    kmax = -(-NCHUNK // nw)
    jmax = -(-kmax // NBUF)

    mesh = plsc.VectorSubcoreMesh(core_axis_name="c", subcore_axis_name="s")

    @functools.partial(
        pl.kernel,
        out_type=jax.ShapeDtypeStruct((ROWS, D_LAT), jnp.float32),
        mesh=mesh,
        scratch_types=[
            pltpu.VMEM((CHUNK,), jnp.int32),
            pltpu.VMEM((CHUNK,), jnp.int32),
            pltpu.VMEM((CHUNK, D_LAT), jnp.float32),
            pltpu.VMEM((CHUNK, D_LAT), jnp.float32),
            pltpu.SemaphoreType.DMA,
            pltpu.SemaphoreType.DMA,
            pltpu.SemaphoreType.DMA,
            pltpu.SemaphoreType.DMA,
        ],
    )
    def gather_kernel(
        table_hbm, idx_hbm, out_hbm, i0, i1, r0, r1, g0, g1, w0, w1
    ):
        wid = lax.axis_index("s") * nc + lax.axis_index("c")
        idx_v = (i0, i1)
        rows_v = (r0, r1)
        gsem = (g0, g1)
        wsem = (w0, w1)

        def body(j, carry):
            for b in range(NBUF):
                c = (j * NBUF + b) * nw + wid

                @pl.when(c < NCHUNK)
                def _(b=b, c=c):
                    @pl.when(j > 0)
                    def _():
                        pltpu.make_async_copy(
                            rows_v[b], out_hbm.at[pl.ds(0, CHUNK)], wsem[b]
                        ).wait()

                    base = c * CHUNK
                    pltpu.sync_copy(idx_hbm.at[pl.ds(base, CHUNK)], idx_v[b])
                    pltpu.async_copy(table_hbm.at[idx_v[b]], rows_v[b], gsem[b])

            for b in range(NBUF):
                c = (j * NBUF + b) * nw + wid

                @pl.when(c < NCHUNK)
                def _(b=b, c=c):
                    pltpu.make_async_copy(
                        table_hbm.at[idx_v[b]], rows_v[b], gsem[b]
                    ).wait()
                    pltpu.async_copy(
                        rows_v[b], out_hbm.at[pl.ds(c * CHUNK, CHUNK)], wsem[b]
                    )

            return carry

        lax.fori_loop(0, jmax, body, 0)

        for b in range(NBUF):
            pltpu.make_async_copy(
                rows_v[b], out_hbm.at[pl.ds(0, CHUNK)], wsem[b]
            ).wait()

    return gather_kernel(table, idx_flat)


def _tc_transpose(x):
    def tkernel(x_ref, o_ref):
        o_ref[...] = x_ref[...].T

    return pl.pallas_call(
        tkernel,
        out_shape=jax.ShapeDtypeStruct((N_VERTEX, N_ANCIL), jnp.float32),
    )(x)


def kernel(z_prime, x_ancil, index):
    idx_flat = index.reshape(ROWS).astype(jnp.int32)
    z_rows = _sc_gather(z_prime, idx_flat)
    z_tilde = z_rows.reshape(N_VERTEX, NU, D_LAT)
    x_ancil_tilde = _tc_transpose(x_ancil)
    return z_tilde, x_ancil_tilde
